# baseline (speedup 1.0000x reference)
import functools

import jax
import jax.numpy as jnp
import numpy as np
from jax import lax
from jax.experimental import pallas as pl
from jax.experimental.pallas import tpu as pltpu
from jax.experimental.pallas import tpu_sc as plsc

_OFFSET = (0.0, 1.0, 1.25, 1.5, 1.75, 2.0, 2.25, 2.5, 2.75, 3.0,
           3.5, 4.0, 4.5, 5.0, 5.5, 6.0, 7.0, 8.0, 9.0, 10.0)
_COEFF = -0.5
_NC = 2
_NS = 16
_NW = _NC * _NS
_GW = 256
_GG = 200
_GS = 200
_ZR = 40


def _silu(v):
    return v * jax.nn.sigmoid(v)


_RND = np.uint32(0x8000)
_HIMASK = np.uint32(0xFFFF0000)


def _pack2(lo_f32, hi_f32):
    lo = (lax.bitcast_convert_type(lo_f32, jnp.uint32) + _RND) >> 16
    hi = (lax.bitcast_convert_type(hi_f32, jnp.uint32) + _RND) & _HIMASK
    return lo | hi


def _unpack_lo(w):
    return lax.bitcast_convert_type(w << 16, jnp.float32)


def _unpack_hi(w):
    return lax.bitcast_convert_type(w & _HIMASK, jnp.float32)


def _prepass_body(h_ref, x_ref, w1a_ref, w1b_ref, b1_ref, t1_ref, t2_ref):
    h = h_ref[...]
    a = jnp.dot(h, w1a_ref[...], preferred_element_type=jnp.float32) + b1_ref[...]
    b = jnp.dot(h, w1b_ref[...], preferred_element_type=jnp.float32)
    x = x_ref[...]
    pad = jnp.zeros((h.shape[0], 125), jnp.float32)
    xc = jnp.concatenate([x, pad], axis=1)
    t1_ref[...] = _pack2(a, xc)
    t2_ref[...] = _pack2(b, -xc)


def _prepass(h, x, w1a, w1b, b1):
    n, hdim = h.shape
    rb = 1000
    return pl.pallas_call(
        _prepass_body,
        grid=(n // rb,),
        in_specs=[
            pl.BlockSpec((rb, hdim), lambda i: (i, 0)),
            pl.BlockSpec((rb, 3), lambda i: (i, 0)),
            pl.BlockSpec((hdim, hdim), lambda i: (0, 0)),
            pl.BlockSpec((hdim, hdim), lambda i: (0, 0)),
            pl.BlockSpec((1, hdim), lambda i: (0, 0)),
        ],
        out_specs=[
            pl.BlockSpec((rb, hdim), lambda i: (i, 0)),
            pl.BlockSpec((rb, hdim), lambda i: (i, 0)),
        ],
        out_shape=[jax.ShapeDtypeStruct((n, hdim), jnp.uint32)] * 2,
    )(h, x, w1a, w1b, b1.reshape(1, hdim))


def _gather_sc(t1, t2, dst, src):
    e = dst.shape[0]
    per_w = e // _NW
    mesh = plsc.VectorSubcoreMesh(core_axis_name="c", subcore_axis_name="s")

    @functools.partial(
        pl.kernel,
        mesh=mesh,
        out_type=(
            jax.ShapeDtypeStruct((e, 128), jnp.uint32),
            jax.ShapeDtypeStruct((e, 128), jnp.uint32),
        ),
        scratch_types=[
            pltpu.VMEM((_GG,), jnp.int32),
            pltpu.VMEM((_GG,), jnp.int32),
            pltpu.VMEM((_GG, 128), jnp.uint32),
            pltpu.VMEM((_GG, 128), jnp.uint32),
        ],
    )
    def k(t1_hbm, t2_hbm, dst_hbm, src_hbm, g1_hbm, g2_hbm,
          idx1, idx2, buf1, buf2):
        c = lax.axis_index("c")
        s = lax.axis_index("s")
        base = (s * _NC + c) * per_w

        @pl.loop(0, per_w, step=_GG)
        def _(off):
            b = base + off
            pltpu.sync_copy(dst_hbm.at[pl.ds(b, _GG)], idx1)
            pltpu.sync_copy(src_hbm.at[pl.ds(b, _GG)], idx2)
            pltpu.sync_copy(t1_hbm.at[idx1], buf1)
            pltpu.sync_copy(t2_hbm.at[idx2], buf2)
            pltpu.sync_copy(buf1, g1_hbm.at[pl.ds(b, _GG)])
            pltpu.sync_copy(buf2, g2_hbm.at[pl.ds(b, _GG)])

    return k(t1, t2, dst, src)


def _edge_body(g1_ref, g2_ref, ea_ref, off_ref, w1de_ref, w2_ref,
               b2_ref, winf_ref, binf_ref, xw1_ref, xb1_ref, xw2_ref,
               s_ref, v_ref):
    w1 = g1_ref[...]
    w2 = g2_ref[...]
    pre = _unpack_lo(w1) + _unpack_lo(w2)
    relx = (_unpack_hi(w1) + _unpack_hi(w2))[:, :3]
    dsq = jnp.sum(relx * relx, axis=1, keepdims=True)
    dist = jnp.sqrt(dsq + 1e-8)
    off = off_ref[...]
    dfeat = jnp.exp(_COEFF * (dist - off) ** 2)
    ef = jnp.concatenate([dfeat, ea_ref[...]], axis=1)
    bf = jnp.bfloat16
    pre = pre + jnp.dot(ef.astype(bf), w1de_ref[...].astype(bf),
                        preferred_element_type=jnp.float32)
    y1 = _silu(pre)
    mij = _silu(jnp.dot(y1.astype(bf), w2_ref[...].astype(bf),
                        preferred_element_type=jnp.float32) + b2_ref[...])
    eij = jax.nn.sigmoid(
        jnp.sum(mij * winf_ref[...], axis=1, keepdims=True) + binf_ref[...])
    t = _silu(jnp.dot(mij.astype(bf), xw1_ref[...].astype(bf),
                      preferred_element_type=jnp.float32) + xb1_ref[...])
    xg = jnp.tanh(jnp.sum(t * xw2_ref[...], axis=1, keepdims=True))
    s_ref[...] = mij * eij
    v = relx * (xg / (dist + 1.0))
    pad = jnp.zeros((v.shape[0], 125), jnp.float32)
    v_ref[...] = jnp.concatenate([v, pad], axis=1)


def _edge_stage(g1, g2, ea, w1de, w2, b2, w_inf, b_inf, xw1, xb1, xw2):
    e = g1.shape[0]
    be = 2000
    nde, hdim = w1de.shape
    return pl.pallas_call(
        _edge_body,
        grid=(e // be,),
        in_specs=[
            pl.BlockSpec((be, hdim), lambda i: (i, 0)),
            pl.BlockSpec((be, hdim), lambda i: (i, 0)),
            pl.BlockSpec((be, ea.shape[1]), lambda i: (i, 0)),
            pl.BlockSpec((1, len(_OFFSET)), lambda i: (0, 0)),
            pl.BlockSpec((nde, hdim), lambda i: (0, 0)),
            pl.BlockSpec((hdim, hdim), lambda i: (0, 0)),
            pl.BlockSpec((1, hdim), lambda i: (0, 0)),
            pl.BlockSpec((1, hdim), lambda i: (0, 0)),
            pl.BlockSpec((1, 1), lambda i: (0, 0)),
            pl.BlockSpec((hdim, hdim), lambda i: (0, 0)),
            pl.BlockSpec((1, hdim), lambda i: (0, 0)),
            pl.BlockSpec((1, hdim), lambda i: (0, 0)),
        ],
        out_specs=[
            pl.BlockSpec((be, hdim), lambda i: (i, 0)),
            pl.BlockSpec((be, hdim), lambda i: (i, 0)),
        ],
        out_shape=[
            jax.ShapeDtypeStruct((e, hdim), jnp.float32),
            jax.ShapeDtypeStruct((e, hdim), jnp.float32),
        ],
    )(g1, g2, ea, jnp.asarray(_OFFSET, jnp.float32).reshape(1, -1),
      w1de, w2, b2.reshape(1, hdim), w_inf.reshape(1, hdim),
      b_inf.reshape(1, 1), xw1, xb1.reshape(1, hdim), xw2.reshape(1, hdim))


def _scatter_sc(sarr, varr, dst, n_nodes):
    e = dst.shape[0]
    hdim = sarr.shape[1]
    per_w = e // _NW
    n_chunks = n_nodes // _ZR
    k_outer = (n_chunks + _NS - 1) // _NS
    mesh = plsc.VectorSubcoreMesh(core_axis_name="c", subcore_axis_name="s")

    @functools.partial(
        pl.kernel,
        mesh=mesh,
        out_type=(
            jax.ShapeDtypeStruct((n_nodes, hdim), jnp.float32),
            jax.ShapeDtypeStruct((n_nodes, hdim), jnp.float32),
            jax.ShapeDtypeStruct((n_nodes, hdim), jnp.float32),
            jax.ShapeDtypeStruct((n_nodes, hdim), jnp.float32),
        ),
        scratch_types=[
            pltpu.VMEM((_GS,), jnp.int32),
            pltpu.VMEM((_GS, 128), jnp.float32),
            pltpu.VMEM((_ZR, 128), jnp.float32),
            pltpu.VMEM_SHARED((n_nodes, 128), jnp.float32),
        ],
    )
    def k(s_hbm, v_hbm, dst_hbm, o0_hbm, o1_hbm, vo0_hbm, vo1_hbm,
          idx, bufs, zbuf, acc):
        c = lax.axis_index("c")
        s = lax.axis_index("s")
        wid = c * _NS + s
        base = wid * per_w
        zv = jnp.zeros((16,), jnp.float32)

        @pl.loop(0, _ZR)
        def _(r):
            @pl.loop(0, 128, step=16)
            def _(c0):
                zbuf.at[r, pl.ds(c0, 16)][...] = zv

        def zero_acc():
            @pl.loop(0, k_outer)
            def _(ko):
                ch = s + ko * _NS

                @pl.when(ch < n_chunks)
                def _():
                    pltpu.sync_copy(zbuf, acc.at[pl.ds(ch * _ZR, _ZR)])

        def scatter_pass(in_hbm):
            @pl.loop(0, per_w, step=_GS)
            def _(off):
                b = base + off
                pltpu.sync_copy(dst_hbm.at[pl.ds(b, _GS)], idx)
                pltpu.sync_copy(in_hbm.at[pl.ds(b, _GS)], bufs)
                pltpu.sync_copy(bufs, acc.at[idx], add=True)

        def writeout(out0_hbm, out1_hbm):
            @pl.loop(0, k_outer)
            def _(ko):
                ch = s + ko * _NS

                @pl.when(ch < n_chunks)
                def _():
                    sl = pl.ds(ch * _ZR, _ZR)

                    @pl.when(c == 0)
                    def _():
                        pltpu.sync_copy(acc.at[sl], out0_hbm.at[sl])

                    @pl.when(c == 1)
                    def _():
                        pltpu.sync_copy(acc.at[sl], out1_hbm.at[sl])

        zero_acc()
        plsc.subcore_barrier()
        scatter_pass(s_hbm)
        plsc.subcore_barrier()
        writeout(o0_hbm, o1_hbm)
        plsc.subcore_barrier()
        zero_acc()
        plsc.subcore_barrier()
        scatter_pass(v_hbm)
        plsc.subcore_barrier()
        writeout(vo0_hbm, vo1_hbm)

    return k(sarr, varr, dst)


def _node_body(m0_ref, m1_ref, vo0_ref, vo1_ref, h_ref, x_ref,
               nw1a_ref, nw1b_ref, nb1_ref, nw2_ref, nb2_ref,
               ho_ref, xo_ref):
    mi = m0_ref[...] + m1_ref[...]
    h = h_ref[...]
    u = _silu(jnp.dot(mi, nw1a_ref[...], preferred_element_type=jnp.float32)
              + jnp.dot(h, nw1b_ref[...], preferred_element_type=jnp.float32)
              + nb1_ref[...])
    ho_ref[...] = h + jnp.dot(u, nw2_ref[...],
                              preferred_element_type=jnp.float32) + nb2_ref[...]
    dx = vo0_ref[...][:, :3] + vo1_ref[...][:, :3]
    xo_ref[...] = x_ref[...] + dx


def _node_stage(m0, m1, vo0, vo1, h, x, nw1a, nw1b, nb1, nw2, nb2):
    n, hdim = h.shape
    rb = 1000
    return pl.pallas_call(
        _node_body,
        grid=(n // rb,),
        in_specs=[
            pl.BlockSpec((rb, hdim), lambda i: (i, 0)),
            pl.BlockSpec((rb, hdim), lambda i: (i, 0)),
            pl.BlockSpec((rb, hdim), lambda i: (i, 0)),
            pl.BlockSpec((rb, hdim), lambda i: (i, 0)),
            pl.BlockSpec((rb, hdim), lambda i: (i, 0)),
            pl.BlockSpec((rb, 3), lambda i: (i, 0)),
            pl.BlockSpec((hdim, hdim), lambda i: (0, 0)),
            pl.BlockSpec((hdim, hdim), lambda i: (0, 0)),
            pl.BlockSpec((1, hdim), lambda i: (0, 0)),
            pl.BlockSpec((hdim, hdim), lambda i: (0, 0)),
            pl.BlockSpec((1, hdim), lambda i: (0, 0)),
        ],
        out_specs=[
            pl.BlockSpec((rb, hdim), lambda i: (i, 0)),
            pl.BlockSpec((rb, 3), lambda i: (i, 0)),
        ],
        out_shape=[
            jax.ShapeDtypeStruct((n, hdim), jnp.float32),
            jax.ShapeDtypeStruct((n, 3), jnp.float32),
        ],
    )(m0, m1, vo0, vo1, h, x, nw1a, nw1b, nb1.reshape(1, hdim), nw2,
      nb2.reshape(1, hdim))


def kernel(h, x, edge_index, mask_ligand, edge_attr, W1, b1, W2, b2,
           w_inf, b_inf, xW1, xb1, xW2, nW1, nb1, nW2, nb2):
    n, hdim = h.shape
    src = edge_index[0]
    dst = edge_index[1]
    t1, t2 = _prepass(h, x, W1[:hdim], W1[hdim:2 * hdim], b1)
    g1, g2 = _gather_sc(t1, t2, dst, src)
    s, v = _edge_stage(g1, g2, edge_attr, W1[2 * hdim:], W2, b2,
                       w_inf, b_inf, xW1, xb1, xW2)
    m0, m1, vo0, vo1 = _scatter_sc(s, v, dst, n)
    h_out, x_out = _node_stage(m0, m1, vo0, vo1, h, x,
                               nW1[:hdim], nW1[hdim:], nb1, nW2, nb2)
    return (h_out, x_out)


# speedup vs baseline: 1.3000x; 1.3000x over previous
import functools

import jax
import jax.numpy as jnp
import numpy as np
from jax import lax
from jax.experimental import pallas as pl
from jax.experimental.pallas import tpu as pltpu
from jax.experimental.pallas import tpu_sc as plsc

_OFFSET = (0.0, 1.0, 1.25, 1.5, 1.75, 2.0, 2.25, 2.5, 2.75, 3.0,
           3.5, 4.0, 4.5, 5.0, 5.5, 6.0, 7.0, 8.0, 9.0, 10.0)
_COEFF = -0.5
_NC = 2
_NS = 16
_NW = _NC * _NS
_GW = 256
_GG = 400
_GS = 200
_ZR = 40


def _silu(v):
    return v * jax.nn.sigmoid(v)


_RND = np.uint32(0x8000)
_HIMASK = np.uint32(0xFFFF0000)


def _pack2(lo_f32, hi_f32):
    lo = (lax.bitcast_convert_type(lo_f32, jnp.uint32) + _RND) >> 16
    hi = (lax.bitcast_convert_type(hi_f32, jnp.uint32) + _RND) & _HIMASK
    return lo | hi


def _unpack_lo(w):
    return lax.bitcast_convert_type(w << 16, jnp.float32)


def _unpack_hi(w):
    return lax.bitcast_convert_type(w & _HIMASK, jnp.float32)


def _prepass_body(h_ref, x_ref, w1a_ref, w1b_ref, b1_ref, t1_ref, t2_ref):
    h = h_ref[...]
    a = jnp.dot(h, w1a_ref[...], preferred_element_type=jnp.float32) + b1_ref[...]
    b = jnp.dot(h, w1b_ref[...], preferred_element_type=jnp.float32)
    x = x_ref[...]
    pad = jnp.zeros((h.shape[0], 125), jnp.float32)
    xc = jnp.concatenate([x, pad], axis=1)
    t1_ref[...] = _pack2(a, xc)
    t2_ref[...] = _pack2(b, -xc)


def _prepass(h, x, w1a, w1b, b1):
    n, hdim = h.shape
    rb = 1000
    return pl.pallas_call(
        _prepass_body,
        grid=(n // rb,),
        in_specs=[
            pl.BlockSpec((rb, hdim), lambda i: (i, 0)),
            pl.BlockSpec((rb, 3), lambda i: (i, 0)),
            pl.BlockSpec((hdim, hdim), lambda i: (0, 0)),
            pl.BlockSpec((hdim, hdim), lambda i: (0, 0)),
            pl.BlockSpec((1, hdim), lambda i: (0, 0)),
        ],
        out_specs=[
            pl.BlockSpec((rb, hdim), lambda i: (i, 0)),
            pl.BlockSpec((rb, hdim), lambda i: (i, 0)),
        ],
        out_shape=[jax.ShapeDtypeStruct((n, hdim), jnp.uint32)] * 2,
    )(h, x, w1a, w1b, b1.reshape(1, hdim))


def _gather_sc(t1, t2, dst, src):
    e = dst.shape[0]
    per_w = e // _NW
    mesh = plsc.VectorSubcoreMesh(core_axis_name="c", subcore_axis_name="s")

    @functools.partial(
        pl.kernel,
        mesh=mesh,
        out_type=(
            jax.ShapeDtypeStruct((e, 128), jnp.uint32),
            jax.ShapeDtypeStruct((e, 128), jnp.uint32),
        ),
        scratch_types=[
            pltpu.VMEM((_GG,), jnp.int32),
            pltpu.VMEM((_GG,), jnp.int32),
            pltpu.VMEM((_GG, 128), jnp.uint32),
            pltpu.VMEM((_GG, 128), jnp.uint32),
            pltpu.SemaphoreType.DMA,
            pltpu.SemaphoreType.DMA,
        ],
    )
    def k(t1_hbm, t2_hbm, dst_hbm, src_hbm, g1_hbm, g2_hbm,
          idx1, idx2, buf1, buf2, sem1, sem2):
        c = lax.axis_index("c")
        s = lax.axis_index("s")
        base = (s * _NC + c) * per_w

        @pl.loop(0, per_w, step=_GG)
        def _(off):
            b = base + off
            pltpu.sync_copy(dst_hbm.at[pl.ds(b, _GG)], idx1)
            pltpu.sync_copy(src_hbm.at[pl.ds(b, _GG)], idx2)
            cp1 = pltpu.async_copy(t1_hbm.at[idx1], buf1, sem1)
            cp2 = pltpu.async_copy(t2_hbm.at[idx2], buf2, sem2)
            cp1.wait()
            cp2.wait()
            pltpu.sync_copy(buf1, g1_hbm.at[pl.ds(b, _GG)])
            pltpu.sync_copy(buf2, g2_hbm.at[pl.ds(b, _GG)])

    return k(t1, t2, dst, src)


def _edge_body(g1_ref, g2_ref, ea_ref, off_ref, w1de_ref, w2_ref,
               b2_ref, winf_ref, binf_ref, xw1_ref, xb1_ref, xw2_ref,
               s_ref, v_ref):
    w1 = g1_ref[...]
    w2 = g2_ref[...]
    pre = _unpack_lo(w1) + _unpack_lo(w2)
    relx = (_unpack_hi(w1) + _unpack_hi(w2))[:, :3]
    dsq = jnp.sum(relx * relx, axis=1, keepdims=True)
    dist = jnp.sqrt(dsq + 1e-8)
    off = off_ref[...]
    dfeat = jnp.exp(_COEFF * (dist - off) ** 2)
    ef = jnp.concatenate([dfeat, ea_ref[...]], axis=1)
    pre = pre + jnp.dot(ef, w1de_ref[...], preferred_element_type=jnp.float32)
    y1 = _silu(pre)
    mij = _silu(jnp.dot(y1, w2_ref[...], preferred_element_type=jnp.float32)
                + b2_ref[...])
    eij = jax.nn.sigmoid(
        jnp.sum(mij * winf_ref[...], axis=1, keepdims=True) + binf_ref[...])
    t = _silu(jnp.dot(mij, xw1_ref[...], preferred_element_type=jnp.float32)
              + xb1_ref[...])
    xg = jnp.tanh(jnp.sum(t * xw2_ref[...], axis=1, keepdims=True))
    s_ref[...] = mij * eij
    v = relx * (xg / (dist + 1.0))
    pad = jnp.zeros((v.shape[0], 125), jnp.float32)
    v_ref[...] = jnp.concatenate([v, pad], axis=1)


def _edge_stage(g1, g2, ea, w1de, w2, b2, w_inf, b_inf, xw1, xb1, xw2):
    e = g1.shape[0]
    be = 2000
    nde, hdim = w1de.shape
    return pl.pallas_call(
        _edge_body,
        grid=(e // be,),
        in_specs=[
            pl.BlockSpec((be, hdim), lambda i: (i, 0)),
            pl.BlockSpec((be, hdim), lambda i: (i, 0)),
            pl.BlockSpec((be, ea.shape[1]), lambda i: (i, 0)),
            pl.BlockSpec((1, len(_OFFSET)), lambda i: (0, 0)),
            pl.BlockSpec((nde, hdim), lambda i: (0, 0)),
            pl.BlockSpec((hdim, hdim), lambda i: (0, 0)),
            pl.BlockSpec((1, hdim), lambda i: (0, 0)),
            pl.BlockSpec((1, hdim), lambda i: (0, 0)),
            pl.BlockSpec((1, 1), lambda i: (0, 0)),
            pl.BlockSpec((hdim, hdim), lambda i: (0, 0)),
            pl.BlockSpec((1, hdim), lambda i: (0, 0)),
            pl.BlockSpec((1, hdim), lambda i: (0, 0)),
        ],
        out_specs=[
            pl.BlockSpec((be, hdim), lambda i: (i, 0)),
            pl.BlockSpec((be, hdim), lambda i: (i, 0)),
        ],
        out_shape=[
            jax.ShapeDtypeStruct((e, hdim), jnp.float32),
            jax.ShapeDtypeStruct((e, hdim), jnp.float32),
        ],
    )(g1, g2, ea, jnp.asarray(_OFFSET, jnp.float32).reshape(1, -1),
      w1de, w2, b2.reshape(1, hdim), w_inf.reshape(1, hdim),
      b_inf.reshape(1, 1), xw1, xb1.reshape(1, hdim), xw2.reshape(1, hdim))


def _scatter_sc(sarr, varr, dst, n_nodes):
    e = dst.shape[0]
    hdim = sarr.shape[1]
    per_w = e // _NW
    n_chunks = n_nodes // _ZR
    k_outer = (n_chunks + _NS - 1) // _NS
    mesh = plsc.VectorSubcoreMesh(core_axis_name="c", subcore_axis_name="s")

    @functools.partial(
        pl.kernel,
        mesh=mesh,
        out_type=(
            jax.ShapeDtypeStruct((n_nodes, hdim), jnp.float32),
            jax.ShapeDtypeStruct((n_nodes, hdim), jnp.float32),
            jax.ShapeDtypeStruct((n_nodes, hdim), jnp.float32),
            jax.ShapeDtypeStruct((n_nodes, hdim), jnp.float32),
        ),
        scratch_types=[
            pltpu.VMEM((_GS,), jnp.int32),
            pltpu.VMEM((_GS, 128), jnp.float32),
            pltpu.VMEM((_ZR, 128), jnp.float32),
            pltpu.VMEM_SHARED((n_nodes, 128), jnp.float32),
        ],
    )
    def k(s_hbm, v_hbm, dst_hbm, o0_hbm, o1_hbm, vo0_hbm, vo1_hbm,
          idx, bufs, zbuf, acc):
        c = lax.axis_index("c")
        s = lax.axis_index("s")
        wid = c * _NS + s
        base = wid * per_w
        zv = jnp.zeros((16,), jnp.float32)

        @pl.loop(0, _ZR)
        def _(r):
            @pl.loop(0, 128, step=16)
            def _(c0):
                zbuf.at[r, pl.ds(c0, 16)][...] = zv

        def zero_acc():
            @pl.loop(0, k_outer)
            def _(ko):
                ch = s + ko * _NS

                @pl.when(ch < n_chunks)
                def _():
                    pltpu.sync_copy(zbuf, acc.at[pl.ds(ch * _ZR, _ZR)])

        def scatter_pass(in_hbm):
            @pl.loop(0, per_w, step=_GS)
            def _(off):
                b = base + off
                pltpu.sync_copy(dst_hbm.at[pl.ds(b, _GS)], idx)
                pltpu.sync_copy(in_hbm.at[pl.ds(b, _GS)], bufs)
                pltpu.sync_copy(bufs, acc.at[idx], add=True)

        def writeout(out0_hbm, out1_hbm):
            @pl.loop(0, k_outer)
            def _(ko):
                ch = s + ko * _NS

                @pl.when(ch < n_chunks)
                def _():
                    sl = pl.ds(ch * _ZR, _ZR)

                    @pl.when(c == 0)
                    def _():
                        pltpu.sync_copy(acc.at[sl], out0_hbm.at[sl])

                    @pl.when(c == 1)
                    def _():
                        pltpu.sync_copy(acc.at[sl], out1_hbm.at[sl])

        zero_acc()
        plsc.subcore_barrier()
        scatter_pass(s_hbm)
        plsc.subcore_barrier()
        writeout(o0_hbm, o1_hbm)
        plsc.subcore_barrier()
        zero_acc()
        plsc.subcore_barrier()
        scatter_pass(v_hbm)
        plsc.subcore_barrier()
        writeout(vo0_hbm, vo1_hbm)

    return k(sarr, varr, dst)


def _node_body(m0_ref, m1_ref, vo0_ref, vo1_ref, h_ref, x_ref,
               nw1a_ref, nw1b_ref, nb1_ref, nw2_ref, nb2_ref,
               ho_ref, xo_ref):
    mi = m0_ref[...] + m1_ref[...]
    h = h_ref[...]
    u = _silu(jnp.dot(mi, nw1a_ref[...], preferred_element_type=jnp.float32)
              + jnp.dot(h, nw1b_ref[...], preferred_element_type=jnp.float32)
              + nb1_ref[...])
    ho_ref[...] = h + jnp.dot(u, nw2_ref[...],
                              preferred_element_type=jnp.float32) + nb2_ref[...]
    dx = vo0_ref[...][:, :3] + vo1_ref[...][:, :3]
    xo_ref[...] = x_ref[...] + dx


def _node_stage(m0, m1, vo0, vo1, h, x, nw1a, nw1b, nb1, nw2, nb2):
    n, hdim = h.shape
    rb = 1000
    return pl.pallas_call(
        _node_body,
        grid=(n // rb,),
        in_specs=[
            pl.BlockSpec((rb, hdim), lambda i: (i, 0)),
            pl.BlockSpec((rb, hdim), lambda i: (i, 0)),
            pl.BlockSpec((rb, hdim), lambda i: (i, 0)),
            pl.BlockSpec((rb, hdim), lambda i: (i, 0)),
            pl.BlockSpec((rb, hdim), lambda i: (i, 0)),
            pl.BlockSpec((rb, 3), lambda i: (i, 0)),
            pl.BlockSpec((hdim, hdim), lambda i: (0, 0)),
            pl.BlockSpec((hdim, hdim), lambda i: (0, 0)),
            pl.BlockSpec((1, hdim), lambda i: (0, 0)),
            pl.BlockSpec((hdim, hdim), lambda i: (0, 0)),
            pl.BlockSpec((1, hdim), lambda i: (0, 0)),
        ],
        out_specs=[
            pl.BlockSpec((rb, hdim), lambda i: (i, 0)),
            pl.BlockSpec((rb, 3), lambda i: (i, 0)),
        ],
        out_shape=[
            jax.ShapeDtypeStruct((n, hdim), jnp.float32),
            jax.ShapeDtypeStruct((n, 3), jnp.float32),
        ],
    )(m0, m1, vo0, vo1, h, x, nw1a, nw1b, nb1.reshape(1, hdim), nw2,
      nb2.reshape(1, hdim))


def kernel(h, x, edge_index, mask_ligand, edge_attr, W1, b1, W2, b2,
           w_inf, b_inf, xW1, xb1, xW2, nW1, nb1, nW2, nb2):
    n, hdim = h.shape
    src = edge_index[0]
    dst = edge_index[1]
    t1, t2 = _prepass(h, x, W1[:hdim], W1[hdim:2 * hdim], b1)
    g1, g2 = _gather_sc(t1, t2, dst, src)
    s, v = _edge_stage(g1, g2, edge_attr, W1[2 * hdim:], W2, b2,
                       w_inf, b_inf, xW1, xb1, xW2)
    m0, m1, vo0, vo1 = _scatter_sc(s, v, dst, n)
    h_out, x_out = _node_stage(m0, m1, vo0, vo1, h, x,
                               nW1[:hdim], nW1[hdim:], nb1, nW2, nb2)
    return (h_out, x_out)
